# baseline (speedup 1.0000x reference)
import functools

import jax
import jax.numpy as jnp
from jax import lax
from jax.experimental import pallas as pl
from jax.experimental.pallas import tpu as pltpu
from jax.experimental.pallas import tpu_sc as plsc

_NC = 2
_NS = 16
_NW = _NC * _NS
_CHUNK = 128
_LANES = 16
_SCALE = 8.0


def _gather_scale_body(table_hbm, idx_hbm, out_hbm, idx_v, rows_v, sem):
    nchunk = idx_v.shape[0]
    d = rows_v.shape[1]
    wid = lax.axis_index("s") * _NC + lax.axis_index("c")
    row_base = wid * (nchunk * _CHUNK)
    pltpu.sync_copy(idx_hbm.at[pl.ds(wid * nchunk, nchunk)], idx_v)

    def chunk(j, carry):
        pltpu.async_copy(table_hbm.at[idx_v.at[j]], rows_v, sem).wait()

        def rowloop(r, c2):
            for c in range(d // _LANES):
                sl = pl.ds(c * _LANES, _LANES)
                rows_v[r, sl] = rows_v[r, sl] * _SCALE
            return c2

        lax.fori_loop(0, _CHUNK, rowloop, 0)
        pltpu.sync_copy(rows_v, out_hbm.at[pl.ds(row_base + j * _CHUNK, _CHUNK)])
        return carry

    lax.fori_loop(0, nchunk, chunk, 0)


def kernel(x, table):
    b, l = x.shape
    _, d = table.shape
    n = b * l
    nchunk = n // (_NW * _CHUNK)
    idx = x.reshape(n // _CHUNK, _CHUNK).astype(jnp.int32)
    mesh = plsc.VectorSubcoreMesh(core_axis_name="c", subcore_axis_name="s")
    f = functools.partial(
        pl.kernel,
        mesh=mesh,
        out_type=jax.ShapeDtypeStruct((n, d), jnp.float32),
        scratch_types=[
            pltpu.VMEM((nchunk, _CHUNK), jnp.int32),
            pltpu.VMEM((_CHUNK, d), jnp.float32),
            pltpu.SemaphoreType.DMA,
        ],
        compiler_params=pltpu.CompilerParams(use_tc_tiling_on_sc=False),
    )(_gather_scale_body)
    out = f(table, idx)
    return out.reshape(b, l, d)
